# Initial kernel scaffold; baseline (speedup 1.0000x reference)
import jax
import jax.numpy as jnp
from jax import lax
from jax.experimental import pallas as pl
from jax.experimental.pallas import tpu as pltpu
from jax.experimental.pallas import tpu_sc as plsc

E = 320000
D = 128
NC = 2
NS = 16
NW = NC * NS
EPW = E // NW
CHUNK = 80
NCHUNK = EPW // CHUNK
GROUPS = CHUNK // 16


def _sc_body(h_hbm, src_hbm, dst_hbm, out_hbm,
             src_idx, dst_idx, src_rows, dst_rows, out_v, sem_s, sem_d):
    wid = lax.axis_index("s") * NC + lax.axis_index("c")
    ebase = pl.multiple_of(wid * EPW, 8)

    pltpu.sync_copy(src_hbm.at[pl.ds(ebase, EPW)], src_idx)
    pltpu.sync_copy(dst_hbm.at[pl.ds(ebase, EPW)], dst_idx)

    def chunk_body(c, carry):
        base = pl.multiple_of(c * CHUNK, 16)
        cp_s = pltpu.make_async_copy(
            h_hbm.at[src_idx.at[pl.ds(base, CHUNK)]], src_rows, sem_s)
        cp_d = pltpu.make_async_copy(
            h_hbm.at[dst_idx.at[pl.ds(base, CHUNK)]], dst_rows, sem_d)
        cp_s.start()
        cp_d.start()
        cp_s.wait()
        cp_d.wait()

        def group_body(g, carry2):
            eidx = g * 16 + lax.iota(jnp.int32, 16)

            def j_body(j8, acc):
                for jj in range(8):
                    jvec = jnp.full((16,), j8 * 8 + jj, jnp.int32)
                    s = plsc.load_gather(src_rows, [eidx, jvec])
                    d = plsc.load_gather(dst_rows, [eidx, jvec])
                    acc = acc + s * d
                return acc

            acc = lax.fori_loop(0, D // 8, j_body, jnp.zeros((16,), jnp.float32))
            out_v[pl.ds(base + g * 16, 16)] = acc
            return carry2

        return lax.fori_loop(0, GROUPS, group_body, carry)

    lax.fori_loop(0, NCHUNK, chunk_body, 0)
    pltpu.sync_copy(out_v, out_hbm.at[pl.ds(ebase, EPW)])


def kernel(h, edge_index):
    ei = edge_index.astype(jnp.int32)
    src = ei[0]
    dst = ei[1]
    mesh = plsc.VectorSubcoreMesh(core_axis_name="c", subcore_axis_name="s")
    out = pl.kernel(
        _sc_body,
        out_type=jax.ShapeDtypeStruct((E,), jnp.float32),
        mesh=mesh,
        scratch_types=[
            pltpu.VMEM((EPW,), jnp.int32),
            pltpu.VMEM((EPW,), jnp.int32),
            pltpu.VMEM((CHUNK, D), jnp.float32),
            pltpu.VMEM((CHUNK, D), jnp.float32),
            pltpu.VMEM((EPW,), jnp.float32),
            pltpu.SemaphoreType.DMA,
            pltpu.SemaphoreType.DMA,
        ],
    )(h, src, dst)
    return out.reshape(E, 1)


# speedup vs baseline: 4.4832x; 4.4832x over previous
import jax
import jax.numpy as jnp
from jax import lax
from jax.experimental import pallas as pl
from jax.experimental.pallas import tpu as pltpu
from jax.experimental.pallas import tpu_sc as plsc

E = 320000
D = 128
NC = 2
NS = 16
NW = NC * NS
EPW = E // NW
CHUNK = 80
NCHUNK = EPW // CHUNK
GROUPS = CHUNK // 16


def _sc_body(h_hbm, src_hbm, dst_hbm, out_hbm,
             src_idx, dst_idx, src_rows, dst_rows, partials, out_v,
             sem_s, sem_d):
    wid = lax.axis_index("s") * NC + lax.axis_index("c")
    ebase = pl.multiple_of(wid * EPW, 8)

    pltpu.sync_copy(src_hbm.at[pl.ds(ebase, EPW)], src_idx)
    pltpu.sync_copy(dst_hbm.at[pl.ds(ebase, EPW)], dst_idx)

    def chunk_body(c, carry):
        base = pl.multiple_of(c * CHUNK, 16)
        cp_s = pltpu.make_async_copy(
            h_hbm.at[src_idx.at[pl.ds(base, CHUNK)]], src_rows, sem_s)
        cp_d = pltpu.make_async_copy(
            h_hbm.at[dst_idx.at[pl.ds(base, CHUNK)]], dst_rows, sem_d)
        cp_s.start()
        cp_d.start()
        cp_s.wait()
        cp_d.wait()

        def edge_body(k, carry2):
            acc = jnp.zeros((16,), jnp.float32)
            for j in range(D // 16):
                s = src_rows[k, pl.ds(j * 16, 16)]
                d = dst_rows[k, pl.ds(j * 16, 16)]
                acc = acc + s * d
            partials[pl.ds(k * 16, 16)] = acc
            return carry2

        lax.fori_loop(0, CHUNK, edge_body, 0)

        def group_body(g, carry2):
            eoff = (g * 16 + lax.iota(jnp.int32, 16)) * 16
            acc = jnp.zeros((16,), jnp.float32)
            for j in range(16):
                acc = acc + plsc.load_gather(partials, [eoff + j])
            out_v[pl.ds(base + g * 16, 16)] = acc
            return carry2

        return lax.fori_loop(0, GROUPS, group_body, carry)

    lax.fori_loop(0, NCHUNK, chunk_body, 0)
    pltpu.sync_copy(out_v, out_hbm.at[pl.ds(ebase, EPW)])


def kernel(h, edge_index):
    ei = edge_index.astype(jnp.int32)
    src = ei[0]
    dst = ei[1]
    mesh = plsc.VectorSubcoreMesh(core_axis_name="c", subcore_axis_name="s")
    out = pl.kernel(
        _sc_body,
        out_type=jax.ShapeDtypeStruct((E,), jnp.float32),
        mesh=mesh,
        compiler_params=pltpu.CompilerParams(needs_layout_passes=False),
        scratch_types=[
            pltpu.VMEM((EPW,), jnp.int32),
            pltpu.VMEM((EPW,), jnp.int32),
            pltpu.VMEM((CHUNK, D), jnp.float32),
            pltpu.VMEM((CHUNK, D), jnp.float32),
            pltpu.VMEM((CHUNK * 16,), jnp.float32),
            pltpu.VMEM((EPW,), jnp.float32),
            pltpu.SemaphoreType.DMA,
            pltpu.SemaphoreType.DMA,
        ],
    )(h, src, dst)
    return out.reshape(E, 1)
